# Initial kernel scaffold; baseline (speedup 1.0000x reference)
import jax
import jax.numpy as jnp
from jax.experimental import pallas as pl


def kernel(pre_box, pre_score, boxes, labels):
    raise NotImplementedError("write your pallas kernel here")


# speedup vs baseline: 62.3216x; 62.3216x over previous
import numpy as np
import jax
import jax.numpy as jnp
from jax.experimental import pallas as pl
from jax.experimental.pallas import tpu as pltpu

_NO_CLASS = 21
_B, _P, _N = 64, 8732, 16


def _make_prior_cxcy():
    fmap_dims = [38, 19, 10, 5, 3, 1]
    obj_scales = [0.1, 0.2, 0.375, 0.55, 0.725, 0.9]
    aspect_ratios = [[1., 2., 0.5], [1., 2., 3., 0.5, .333], [1., 2., 3., 0.5, .333],
                     [1., 2., 3., 0.5, .333], [1., 2., 0.5], [1., 2., 0.5]]
    priors = []
    for k, dim in enumerate(fmap_dims):
        s = obj_scales[k]
        for i in range(dim):
            for j in range(dim):
                cx = (j + 0.5) / dim
                cy = (i + 0.5) / dim
                for ratio in aspect_ratios[k]:
                    priors.append([cx, cy, s * np.sqrt(ratio), s / np.sqrt(ratio)])
                    if ratio == 1.:
                        if k + 1 < len(obj_scales):
                            add_s = np.sqrt(s * obj_scales[k + 1])
                        else:
                            add_s = 1.
                        priors.append([cx, cy, add_s, add_s])
    return np.clip(np.array(priors, dtype=np.float32), 0., 1.)


_PRIOR_CXCY = _make_prior_cxcy()
_PRIOR_XY = np.concatenate(
    [_PRIOR_CXCY[:, :2] - _PRIOR_CXCY[:, 2:] / 2.,
     _PRIOR_CXCY[:, :2] + _PRIOR_CXCY[:, 2:] / 2.], axis=1).astype(np.float32)
_PXY_T = np.ascontiguousarray(_PRIOR_XY.T)
_PCC_T = np.ascontiguousarray(_PRIOR_CXCY.T)


def _match_body(boxes_ref, labels_ref, pb_ref, ps_ref, pxy_ref, pcc_ref,
                cn_ref, npos_ref, cpos_ref, sl1_ref):
    f32 = jnp.float32
    bx = boxes_ref[0]
    x1, y1, x2, y2 = bx[:, 0:1], bx[:, 1:2], bx[:, 2:3], bx[:, 3:4]
    px1 = pxy_ref[0:1, :]
    py1 = pxy_ref[1:2, :]
    px2 = pxy_ref[2:3, :]
    py2 = pxy_ref[3:4, :]

    inter = (jnp.maximum(jnp.minimum(x2, px2) - jnp.maximum(x1, px1), 0.) *
             jnp.maximum(jnp.minimum(y2, py2) - jnp.maximum(y1, py1), 0.))
    area1 = (x2 - x1) * (y2 - y1)
    area2 = (px2 - px1) * (py2 - py1)
    ov = inter / (area1 + area2 - inter)

    ji = jax.lax.broadcasted_iota(jnp.int32, (_N, _P), 0)
    pi = jax.lax.broadcasted_iota(jnp.int32, (_N, _P), 1)
    big = jnp.int32(2 ** 30)

    ov_max = jnp.max(ov, axis=0, keepdims=True)
    obj_fp = jnp.min(jnp.where(ov == ov_max, ji, big), axis=0, keepdims=True)
    ov_omax = jnp.max(ov, axis=1, keepdims=True)
    prior_fo = jnp.min(jnp.where(ov == ov_omax, pi, big), axis=1, keepdims=True)

    j_sel = jnp.max(jnp.where(prior_fo == pi, ji, -1), axis=0, keepdims=True)
    forced = j_sel >= 0
    obj_sel = jnp.where(forced, j_sel, obj_fp)
    ov_sel = jnp.where(forced, f32(1.0), ov_max)

    hit = obj_sel == ji
    labs = labels_ref[0]
    lab_p = jnp.sum(jnp.where(hit, labs, 0), axis=0, keepdims=True)
    true_cls = jnp.where(ov_sel < 0.5, 0, lab_p)
    pos = true_cls != 0
    posf = pos.astype(f32)

    mx1 = jnp.sum(jnp.where(hit, x1, 0.), axis=0, keepdims=True)
    my1 = jnp.sum(jnp.where(hit, y1, 0.), axis=0, keepdims=True)
    mx2 = jnp.sum(jnp.where(hit, x2, 0.), axis=0, keepdims=True)
    my2 = jnp.sum(jnp.where(hit, y2, 0.), axis=0, keepdims=True)
    cx = (mx1 + mx2) / 2.
    cy = (my1 + my2) / 2.
    w = mx2 - mx1
    h = my2 - my1
    pcx = pcc_ref[0:1, :]
    pcy = pcc_ref[1:2, :]
    pw = pcc_ref[2:3, :]
    ph = pcc_ref[3:4, :]
    gcx = (cx - pcx) / (pw / 10.)
    gcy = (cy - pcy) / (ph / 10.)
    gw = jnp.log(w / pw) * 5.
    gh = jnp.log(h / ph) * 5.

    pb = pb_ref[0]
    s_acc = jnp.zeros((1, _P), f32)
    for c, tl in enumerate((gcx, gcy, gw, gh)):
        d = pb[c:c + 1, :] - tl
        ad = jnp.abs(d)
        s_acc = s_acc + jnp.where(ad < 1., 0.5 * d * d, ad - 0.5)
    sl1_sum = jnp.sum(s_acc * posf)

    s = ps_ref[0]
    m = jnp.max(s, axis=0, keepdims=True)
    sh = s - m
    logse = jnp.log(jnp.sum(jnp.exp(sh), axis=0, keepdims=True))
    ci = jax.lax.broadcasted_iota(jnp.int32, (_NO_CLASS, _P), 0)
    sh_at = jnp.sum(jnp.where(true_cls == ci, sh, 0.), axis=0, keepdims=True)
    conf = logse - sh_at

    cn_ref[0] = jnp.where(pos, 0., conf)
    npos_ref[0] = jnp.full((1, 128), jnp.sum(posf), f32)
    cpos_ref[0] = jnp.full((1, 128), jnp.sum(conf * posf), f32)
    sl1_ref[0] = jnp.full((1, 128), sl1_sum, f32)


def _topk_body(cn_ref, npos_ref, cpos_ref, sl1_ref, out_ref):
    x = cn_ref[...]
    npos = npos_ref[:, 0:1]
    k = 3. * npos
    xb = jax.lax.bitcast_convert_type(x, jnp.int32)

    lo0 = jnp.zeros((_B, 1), jnp.int32)
    hi0 = jnp.full((_B, 1), jnp.int32(0x7F800000))

    def step(_, carry):
        lo, hi = carry
        mid = lo + jax.lax.shift_right_logical(hi - lo, 1)
        cnt = jnp.sum(jnp.where(xb > mid, 1., 0.), axis=1, keepdims=True)
        pred = cnt >= k
        return jnp.where(pred, mid + 1, lo), jnp.where(pred, hi, mid)

    lo, _ = jax.lax.fori_loop(0, 31, step, (lo0, hi0))
    t = jax.lax.bitcast_convert_type(lo, jnp.float32)

    gt = x > t
    cnt_gt = jnp.sum(jnp.where(gt, 1., 0.), axis=1, keepdims=True)
    sum_gt = jnp.sum(jnp.where(gt, x, 0.), axis=1, keepdims=True)
    hard = jnp.where(k > 0., sum_gt + (k - cnt_gt) * t, 0.)

    npos_tot = jnp.sum(npos)
    conf_loss = (jnp.sum(cpos_ref[:, 0:1]) + jnp.sum(hard)) / npos_tot
    loc_loss = jnp.sum(sl1_ref[:, 0:1]) / (4. * npos_tot)
    out_ref[...] = jnp.full((1, 128), conf_loss + loc_loss, jnp.float32)


def kernel(pre_box, pre_score, boxes, labels):
    pb_t = jnp.transpose(pre_box, (0, 2, 1))
    ps_t = jnp.transpose(pre_score, (0, 2, 1))
    labels3 = labels.reshape(_B, _N, 1)
    pxy = jnp.asarray(_PXY_T)
    pcc = jnp.asarray(_PCC_T)

    cn, npos, cpos, sl1 = pl.pallas_call(
        _match_body,
        grid=(_B,),
        in_specs=[
            pl.BlockSpec((1, _N, 4), lambda i: (i, 0, 0)),
            pl.BlockSpec((1, _N, 1), lambda i: (i, 0, 0)),
            pl.BlockSpec((1, 4, _P), lambda i: (i, 0, 0)),
            pl.BlockSpec((1, _NO_CLASS, _P), lambda i: (i, 0, 0)),
            pl.BlockSpec((4, _P), lambda i: (0, 0)),
            pl.BlockSpec((4, _P), lambda i: (0, 0)),
        ],
        out_specs=[
            pl.BlockSpec((1, 1, _P), lambda i: (i, 0, 0)),
            pl.BlockSpec((1, 1, 128), lambda i: (i, 0, 0)),
            pl.BlockSpec((1, 1, 128), lambda i: (i, 0, 0)),
            pl.BlockSpec((1, 1, 128), lambda i: (i, 0, 0)),
        ],
        out_shape=[
            jax.ShapeDtypeStruct((_B, 1, _P), jnp.float32),
            jax.ShapeDtypeStruct((_B, 1, 128), jnp.float32),
            jax.ShapeDtypeStruct((_B, 1, 128), jnp.float32),
            jax.ShapeDtypeStruct((_B, 1, 128), jnp.float32),
        ],
        compiler_params=pltpu.CompilerParams(
            dimension_semantics=("arbitrary",)),
    )(boxes, labels3, pb_t, ps_t, pxy, pcc)

    out = pl.pallas_call(
        _topk_body,
        out_shape=jax.ShapeDtypeStruct((1, 128), jnp.float32),
    )(cn.reshape(_B, _P), npos.reshape(_B, 128),
      cpos.reshape(_B, 128), sl1.reshape(_B, 128))
    return out[0, 0]


# speedup vs baseline: 62.3883x; 1.0011x over previous
import numpy as np
import jax
import jax.numpy as jnp
from jax.experimental import pallas as pl
from jax.experimental.pallas import tpu as pltpu

_NO_CLASS = 21
_B, _P, _N = 64, 8732, 16


def _make_prior_cxcy():
    fmap_dims = [38, 19, 10, 5, 3, 1]
    obj_scales = [0.1, 0.2, 0.375, 0.55, 0.725, 0.9]
    aspect_ratios = [[1., 2., 0.5], [1., 2., 3., 0.5, .333], [1., 2., 3., 0.5, .333],
                     [1., 2., 3., 0.5, .333], [1., 2., 0.5], [1., 2., 0.5]]
    priors = []
    for k, dim in enumerate(fmap_dims):
        s = obj_scales[k]
        for i in range(dim):
            for j in range(dim):
                cx = (j + 0.5) / dim
                cy = (i + 0.5) / dim
                for ratio in aspect_ratios[k]:
                    priors.append([cx, cy, s * np.sqrt(ratio), s / np.sqrt(ratio)])
                    if ratio == 1.:
                        if k + 1 < len(obj_scales):
                            add_s = np.sqrt(s * obj_scales[k + 1])
                        else:
                            add_s = 1.
                        priors.append([cx, cy, add_s, add_s])
    return np.clip(np.array(priors, dtype=np.float32), 0., 1.)


_PRIOR_CXCY = _make_prior_cxcy()
_PRIOR_XY = np.concatenate(
    [_PRIOR_CXCY[:, :2] - _PRIOR_CXCY[:, 2:] / 2.,
     _PRIOR_CXCY[:, :2] + _PRIOR_CXCY[:, 2:] / 2.], axis=1).astype(np.float32)
_PXY_T = np.ascontiguousarray(_PRIOR_XY.T)
_PCC_T = np.ascontiguousarray(_PRIOR_CXCY.T)


def _match_body(boxes_ref, labels_ref, pb_ref, ps_ref, pxy_ref, pcc_ref,
                cn_ref, npos_ref, cpos_ref, sl1_ref):
    f32 = jnp.float32
    bx = boxes_ref[0]
    x1, y1, x2, y2 = bx[:, 0:1], bx[:, 1:2], bx[:, 2:3], bx[:, 3:4]
    px1 = pxy_ref[0:1, :]
    py1 = pxy_ref[1:2, :]
    px2 = pxy_ref[2:3, :]
    py2 = pxy_ref[3:4, :]

    inter = (jnp.maximum(jnp.minimum(x2, px2) - jnp.maximum(x1, px1), 0.) *
             jnp.maximum(jnp.minimum(y2, py2) - jnp.maximum(y1, py1), 0.))
    area1 = (x2 - x1) * (y2 - y1)
    area2 = (px2 - px1) * (py2 - py1)
    ov = inter / (area1 + area2 - inter)

    ji = jax.lax.broadcasted_iota(jnp.int32, (_N, _P), 0)
    pi = jax.lax.broadcasted_iota(jnp.int32, (_N, _P), 1)
    big = jnp.int32(2 ** 30)

    ov_max = jnp.max(ov, axis=0, keepdims=True)
    obj_fp = jnp.min(jnp.where(ov == ov_max, ji, big), axis=0, keepdims=True)
    ov_omax = jnp.max(ov, axis=1, keepdims=True)
    prior_fo = jnp.min(jnp.where(ov == ov_omax, pi, big), axis=1, keepdims=True)

    j_sel = jnp.max(jnp.where(prior_fo == pi, ji, -1), axis=0, keepdims=True)
    forced = j_sel >= 0
    obj_sel = jnp.where(forced, j_sel, obj_fp)
    ov_sel = jnp.where(forced, f32(1.0), ov_max)

    hit = obj_sel == ji
    labs = labels_ref[0]
    lab_p = jnp.sum(jnp.where(hit, labs, 0), axis=0, keepdims=True)
    true_cls = jnp.where(ov_sel < 0.5, 0, lab_p)
    pos = true_cls != 0
    posf = pos.astype(f32)

    mx1 = jnp.sum(jnp.where(hit, x1, 0.), axis=0, keepdims=True)
    my1 = jnp.sum(jnp.where(hit, y1, 0.), axis=0, keepdims=True)
    mx2 = jnp.sum(jnp.where(hit, x2, 0.), axis=0, keepdims=True)
    my2 = jnp.sum(jnp.where(hit, y2, 0.), axis=0, keepdims=True)
    cx = (mx1 + mx2) / 2.
    cy = (my1 + my2) / 2.
    w = mx2 - mx1
    h = my2 - my1
    pcx = pcc_ref[0:1, :]
    pcy = pcc_ref[1:2, :]
    pw = pcc_ref[2:3, :]
    ph = pcc_ref[3:4, :]
    gcx = (cx - pcx) / (pw / 10.)
    gcy = (cy - pcy) / (ph / 10.)
    gw = jnp.log(w / pw) * 5.
    gh = jnp.log(h / ph) * 5.

    pb = pb_ref[0]
    s_acc = jnp.zeros((1, _P), f32)
    for c, tl in enumerate((gcx, gcy, gw, gh)):
        d = pb[c:c + 1, :] - tl
        ad = jnp.abs(d)
        s_acc = s_acc + jnp.where(ad < 1., 0.5 * d * d, ad - 0.5)
    sl1_sum = jnp.sum(s_acc * posf)

    s = ps_ref[0]
    m = jnp.max(s, axis=0, keepdims=True)
    sh = s - m
    logse = jnp.log(jnp.sum(jnp.exp(sh), axis=0, keepdims=True))
    ci = jax.lax.broadcasted_iota(jnp.int32, (_NO_CLASS, _P), 0)
    sh_at = jnp.sum(jnp.where(true_cls == ci, sh, 0.), axis=0, keepdims=True)
    conf = logse - sh_at

    cn_ref[0] = jnp.where(pos, 0., conf)
    npos_ref[0] = jnp.full((1, 128), jnp.sum(posf), f32)
    cpos_ref[0] = jnp.full((1, 128), jnp.sum(conf * posf), f32)
    sl1_ref[0] = jnp.full((1, 128), sl1_sum, f32)


def _topk_body(cn_ref, npos_ref, cpos_ref, sl1_ref, out_ref):
    x = cn_ref[...]
    npos = npos_ref[:, 0:1]
    k = 3. * npos
    xb = jax.lax.bitcast_convert_type(x, jnp.int32)

    lo0 = jnp.zeros((_B, 1), jnp.int32)
    hi0 = jnp.full((_B, 1), jnp.int32(0x7F800000))

    def step(_, carry):
        lo, hi = carry
        mid = lo + jax.lax.shift_right_logical(hi - lo, 1)
        cnt = jnp.sum(jnp.where(xb > mid, 1., 0.), axis=1, keepdims=True)
        pred = cnt >= k
        return jnp.where(pred, mid + 1, lo), jnp.where(pred, hi, mid)

    lo, _ = jax.lax.fori_loop(0, 31, step, (lo0, hi0))
    t = jax.lax.bitcast_convert_type(lo, jnp.float32)

    gt = x > t
    cnt_gt = jnp.sum(jnp.where(gt, 1., 0.), axis=1, keepdims=True)
    sum_gt = jnp.sum(jnp.where(gt, x, 0.), axis=1, keepdims=True)
    hard = jnp.where(k > 0., sum_gt + (k - cnt_gt) * t, 0.)

    npos_tot = jnp.sum(npos)
    conf_loss = (jnp.sum(cpos_ref[:, 0:1]) + jnp.sum(hard)) / npos_tot
    loc_loss = jnp.sum(sl1_ref[:, 0:1]) / (4. * npos_tot)
    out_ref[...] = jnp.full((1, 128), conf_loss + loc_loss, jnp.float32)


def kernel(pre_box, pre_score, boxes, labels):
    pb_t = jnp.transpose(pre_box, (0, 2, 1))
    ps_t = jnp.transpose(pre_score, (0, 2, 1))
    labels3 = labels.reshape(_B, _N, 1)
    pxy = jnp.asarray(_PXY_T)
    pcc = jnp.asarray(_PCC_T)

    cn, npos, cpos, sl1 = pl.pallas_call(
        _match_body,
        grid=(_B,),
        in_specs=[
            pl.BlockSpec((1, _N, 4), lambda i: (i, 0, 0)),
            pl.BlockSpec((1, _N, 1), lambda i: (i, 0, 0)),
            pl.BlockSpec((1, 4, _P), lambda i: (i, 0, 0)),
            pl.BlockSpec((1, _NO_CLASS, _P), lambda i: (i, 0, 0)),
            pl.BlockSpec((4, _P), lambda i: (0, 0)),
            pl.BlockSpec((4, _P), lambda i: (0, 0)),
        ],
        out_specs=[
            pl.BlockSpec((1, 1, _P), lambda i: (i, 0, 0)),
            pl.BlockSpec((1, 1, 128), lambda i: (i, 0, 0)),
            pl.BlockSpec((1, 1, 128), lambda i: (i, 0, 0)),
            pl.BlockSpec((1, 1, 128), lambda i: (i, 0, 0)),
        ],
        out_shape=[
            jax.ShapeDtypeStruct((_B, 1, _P), jnp.float32),
            jax.ShapeDtypeStruct((_B, 1, 128), jnp.float32),
            jax.ShapeDtypeStruct((_B, 1, 128), jnp.float32),
            jax.ShapeDtypeStruct((_B, 1, 128), jnp.float32),
        ],
        compiler_params=pltpu.CompilerParams(
            dimension_semantics=("parallel",)),
    )(boxes, labels3, pb_t, ps_t, pxy, pcc)

    out = pl.pallas_call(
        _topk_body,
        out_shape=jax.ShapeDtypeStruct((1, 128), jnp.float32),
    )(cn.reshape(_B, _P), npos.reshape(_B, 128),
      cpos.reshape(_B, 128), sl1.reshape(_B, 128))
    return out[0, 0]
